# Initial kernel scaffold; baseline (speedup 1.0000x reference)
import jax
import jax.numpy as jnp
from jax.experimental import pallas as pl


def kernel(x, weights, bias, nonzero_indices):
    raise NotImplementedError("write your pallas kernel here")


# speedup vs baseline: 1.5956x; 1.5956x over previous
import functools

import jax
import jax.numpy as jnp
from jax import lax
from jax.experimental import pallas as pl
from jax.experimental.pallas import tpu as pltpu
from jax.experimental.pallas import tpu_sc as plsc

IN_DIM = 4096
OUT_DIM = 4096
BATCH = 64
NQ = 4
NG = 8
GCOLS = OUT_DIM // NG
QROWS = BATCH // NQ
CH = 2048
LANES = 16


def _sc_body(k_hbm, w_hbm, x_hbm, b_hbm, off_hbm, out_hbm,
             xq, acc, kb, wbuf, bt, offv):
    cidx = lax.axis_index("c")
    sidx = lax.axis_index("s")
    wid = sidx * 2 + cidx
    q = wid & 3
    g = wid >> 2
    g512 = g * GCOLS

    pltpu.sync_copy(x_hbm.at[pl.ds(q * QROWS, QROWS), :], xq)
    pltpu.sync_copy(b_hbm.at[pl.ds(g512, GCOLS)], bt)
    pltpu.sync_copy(off_hbm, offv)

    def init_body(v, carry):
        bv = bt[pl.ds(v * LANES, LANES)]
        for b in range(QROWS):
            acc[b, pl.ds(v * LANES, LANES)] = bv
        return carry
    lax.fori_loop(0, GCOLS // LANES, init_body, 0)

    iota = lax.iota(jnp.int32, LANES)
    ov = offv[...]
    s0 = jnp.sum(jnp.where(iota == g, ov, 0))
    s1 = jnp.sum(jnp.where(iota == g + 1, ov, 0))
    k0 = s0 // CH
    k1 = (s1 + CH - 1) // CH

    def chunk_body(i, carry):
        pltpu.sync_copy(k_hbm.at[pl.ds(i * CH, CH)], kb)
        pltpu.sync_copy(w_hbm.at[pl.ds(i * CH, CH)], wbuf)
        base_i = i * CH

        def group_body(j, gcarry):
            sl = pl.ds(j * LANES, LANES)
            kv = kb[sl]
            wv = wbuf[sl]
            rv = jnp.bitwise_and(kv, IN_DIM - 1)
            cv = jnp.right_shift(kv, 12)
            posv = base_i + j * LANES + iota
            valid = (posv >= s0) & (posv < s1)
            wvm = jnp.where(valid, wv, 0.0)
            clv = jnp.where(valid, cv - g512, 0)
            for b in range(QROWS):
                bvec = jnp.bitwise_and(iota + b, QROWS - 1)
                vals = plsc.load_gather(xq, [bvec, rv])
                plsc.addupdate_scatter(acc, [bvec, clv], vals * wvm)
            return gcarry
        lax.fori_loop(0, CH // LANES, group_body, 0)
        return carry
    lax.fori_loop(k0, k1, chunk_body, 0)

    pltpu.sync_copy(acc, out_hbm.at[pl.ds(q * QROWS, QROWS), pl.ds(g512, GCOLS)])


def _make_sc_kernel(interpret=False):
    mesh = plsc.VectorSubcoreMesh(core_axis_name="c", subcore_axis_name="s")
    return functools.partial(
        pl.kernel,
        out_type=jax.ShapeDtypeStruct((BATCH, OUT_DIM), jnp.float32),
        mesh=mesh,
        scratch_types=[
            pltpu.VMEM((QROWS, IN_DIM), jnp.float32),
            pltpu.VMEM((QROWS, GCOLS), jnp.float32),
            pltpu.VMEM((CH,), jnp.int32),
            pltpu.VMEM((CH,), jnp.float32),
            pltpu.VMEM((GCOLS,), jnp.float32),
            pltpu.VMEM((16,), jnp.int32),
        ],
        compiler_params=pltpu.CompilerParams(use_tc_tiling_on_sc=False,
                                             needs_layout_passes=False),
        interpret=interpret,
    )(_sc_body)


def kernel(x, weights, bias, nonzero_indices):
    nnz = weights.shape[0]
    r = nonzero_indices[:, 0].astype(jnp.int32)
    c = nonzero_indices[:, 1].astype(jnp.int32)
    rkey = r * OUT_DIM + c
    rks, ws = lax.sort((rkey, weights), num_keys=1, is_stable=False)
    keep = jnp.concatenate([rks[1:] != rks[:-1], jnp.ones((1,), bool)])
    ws = jnp.where(keep, ws, 0.0)
    ckey = jnp.bitwise_and(rks, OUT_DIM - 1) * IN_DIM + jnp.right_shift(rks, 12)
    ks, ws = lax.sort((ckey, ws), num_keys=1, is_stable=False)
    offs = jnp.searchsorted(ks, jnp.arange(0, NG + 1, dtype=jnp.int32) * (GCOLS * IN_DIM))
    offs16 = jnp.zeros((16,), jnp.int32).at[:NG + 1].set(offs.astype(jnp.int32))
    nnz_pad = ((nnz + CH - 1) // CH) * CH
    padn = nnz_pad - nnz
    ks = jnp.pad(ks, (0, padn))
    ws = jnp.pad(ws, (0, padn))
    return _make_sc_kernel()(ks, ws, x, bias, offs16)


# speedup vs baseline: 1.8986x; 1.1899x over previous
import functools

import jax
import jax.numpy as jnp
from jax import lax
from jax.experimental import pallas as pl
from jax.experimental.pallas import tpu as pltpu
from jax.experimental.pallas import tpu_sc as plsc

IN_DIM = 4096
OUT_DIM = 4096
BATCH = 64
NQ = 4
NG = 8
GCOLS = OUT_DIM // NG
QROWS = BATCH // NQ
CH = 2048
LANES = 16


def _sc_body(k_hbm, w_hbm, x_hbm, b_hbm, off_hbm, out_hbm,
             xq, acc, kb, wbuf, bt, offv):
    cidx = lax.axis_index("c")
    sidx = lax.axis_index("s")
    wid = sidx * 2 + cidx
    q = wid & 3
    g = wid >> 2
    g512 = g * GCOLS

    pltpu.sync_copy(x_hbm.at[pl.ds(q * QROWS, QROWS), :], xq)
    pltpu.sync_copy(b_hbm.at[pl.ds(g512, GCOLS)], bt)
    pltpu.sync_copy(off_hbm, offv)

    def init_body(v, carry):
        bv = bt[pl.ds(v * LANES, LANES)]
        for b in range(QROWS):
            acc[b, pl.ds(v * LANES, LANES)] = bv
        return carry
    lax.fori_loop(0, GCOLS // LANES, init_body, 0)

    iota = lax.iota(jnp.int32, LANES)
    ov = offv[...]
    s0 = jnp.sum(jnp.where(iota == g, ov, 0))
    s1 = jnp.sum(jnp.where(iota == g + 1, ov, 0))
    k0 = s0 // CH
    k1 = (s1 + CH - 1) // CH

    def chunk_body(i, carry):
        pltpu.sync_copy(k_hbm.at[pl.ds(i * CH, CH)], kb)
        pltpu.sync_copy(w_hbm.at[pl.ds(i * CH, CH)], wbuf)
        base_i = i * CH

        @plsc.parallel_loop(0, CH // LANES, unroll=2)
        def group_body(j):
            sl = pl.ds(j * LANES, LANES)
            kv = kb[sl]
            wv = wbuf[sl]
            rv = jnp.bitwise_and(kv, IN_DIM - 1)
            cv = jnp.right_shift(kv, 12)
            posv = base_i + j * LANES + iota
            valid = (posv >= s0) & (posv < s1)
            wvm = jnp.where(valid, wv, 0.0)
            clv = jnp.where(valid, cv - g512, 0)
            for b in range(QROWS):
                bvec = jnp.bitwise_and(iota + b, QROWS - 1)
                vals = plsc.load_gather(xq, [bvec, rv])
                plsc.addupdate_scatter(acc, [bvec, clv], vals * wvm)
        return carry
    lax.fori_loop(k0, k1, chunk_body, 0)

    pltpu.sync_copy(acc, out_hbm.at[pl.ds(q * QROWS, QROWS), pl.ds(g512, GCOLS)])


def _make_sc_kernel(interpret=False):
    mesh = plsc.VectorSubcoreMesh(core_axis_name="c", subcore_axis_name="s")
    return functools.partial(
        pl.kernel,
        out_type=jax.ShapeDtypeStruct((BATCH, OUT_DIM), jnp.float32),
        mesh=mesh,
        scratch_types=[
            pltpu.VMEM((QROWS, IN_DIM), jnp.float32),
            pltpu.VMEM((QROWS, GCOLS), jnp.float32),
            pltpu.VMEM((CH,), jnp.int32),
            pltpu.VMEM((CH,), jnp.float32),
            pltpu.VMEM((GCOLS,), jnp.float32),
            pltpu.VMEM((16,), jnp.int32),
        ],
        compiler_params=pltpu.CompilerParams(use_tc_tiling_on_sc=False,
                                             needs_layout_passes=False),
        interpret=interpret,
    )(_sc_body)


def kernel(x, weights, bias, nonzero_indices):
    nnz = weights.shape[0]
    r = nonzero_indices[:, 0].astype(jnp.int32)
    c = nonzero_indices[:, 1].astype(jnp.int32)
    rkey = r * OUT_DIM + c
    rks, ws = lax.sort((rkey, weights), num_keys=1, is_stable=False)
    keep = jnp.concatenate([rks[1:] != rks[:-1], jnp.ones((1,), bool)])
    ws = jnp.where(keep, ws, 0.0)
    ckey = jnp.bitwise_and(rks, OUT_DIM - 1) * IN_DIM + jnp.right_shift(rks, 12)
    ks, ws = lax.sort((ckey, ws), num_keys=1, is_stable=False)
    offs = jnp.searchsorted(ks, jnp.arange(0, NG + 1, dtype=jnp.int32) * (GCOLS * IN_DIM))
    offs16 = jnp.zeros((16,), jnp.int32).at[:NG + 1].set(offs.astype(jnp.int32))
    nnz_pad = ((nnz + CH - 1) // CH) * CH
    padn = nnz_pad - nnz
    ks = jnp.pad(ks, (0, padn))
    ws = jnp.pad(ws, (0, padn))
    return _make_sc_kernel()(ks, ws, x, bias, offs16)


# speedup vs baseline: 2.8991x; 1.5270x over previous
import functools

import jax
import jax.numpy as jnp
from jax import lax
from jax.experimental import pallas as pl
from jax.experimental.pallas import tpu as pltpu
from jax.experimental.pallas import tpu_sc as plsc

IN_DIM = 4096
OUT_DIM = 4096
BATCH = 64
NQ = 4
NG = 8
GCOLS = OUT_DIM // NG
QROWS = BATCH // NQ
CH = 2048
LANES = 16


def _sc_body(k_hbm, w_hbm, xt_hbm, b_hbm, off_hbm, out_hbm,
             xqt, acc, kb, wbuf, bt, offv):
    cidx = lax.axis_index("c")
    sidx = lax.axis_index("s")
    wid = sidx * 2 + cidx
    q = wid & 3
    g = wid >> 2
    g512 = g * GCOLS

    pltpu.sync_copy(xt_hbm.at[:, pl.ds(q * QROWS, QROWS)], xqt)
    pltpu.sync_copy(b_hbm.at[pl.ds(g512, GCOLS)], bt)
    pltpu.sync_copy(off_hbm, offv)

    def init_body(v, carry):
        bv = bt[pl.ds(v * LANES, LANES)]
        for l in range(LANES):
            acc[v * LANES + l, :] = jnp.full((LANES,), bv[l], jnp.float32)
        return carry
    lax.fori_loop(0, GCOLS // LANES, init_body, 0)

    iota = lax.iota(jnp.int32, LANES)
    ov = offv[...]
    s0 = jnp.sum(jnp.where(iota == g, ov, 0))
    s1 = jnp.sum(jnp.where(iota == g + 1, ov, 0))
    k0 = s0 // CH
    k1 = (s1 + CH - 1) // CH

    def chunk_body(i, carry):
        pltpu.sync_copy(k_hbm.at[pl.ds(i * CH, CH)], kb)
        pltpu.sync_copy(w_hbm.at[pl.ds(i * CH, CH)], wbuf)
        base_i = i * CH

        @plsc.parallel_loop(0, CH // LANES, unroll=2)
        def group_body(j):
            sl = pl.ds(j * LANES, LANES)
            kv = kb[sl]
            wv = wbuf[sl]
            rv = jnp.bitwise_and(kv, IN_DIM - 1)
            cv = jnp.right_shift(kv, 12)
            posv = base_i + j * LANES + iota
            valid = (posv >= s0) & (posv < s1)
            wvm = jnp.where(valid, wv, 0.0)
            clv = jnp.where(valid, cv - g512, 0)
            for l in range(LANES):
                xrow = xqt[rv[l], :]
                plsc.addupdate(acc.at[clv[l]], xrow * wvm[l])
        return carry
    lax.fori_loop(k0, k1, chunk_body, 0)

    pltpu.sync_copy(acc, out_hbm.at[pl.ds(g512, GCOLS), pl.ds(q * QROWS, QROWS)])


def _make_sc_kernel(interpret=False):
    mesh = plsc.VectorSubcoreMesh(core_axis_name="c", subcore_axis_name="s")
    return functools.partial(
        pl.kernel,
        out_type=jax.ShapeDtypeStruct((OUT_DIM, BATCH), jnp.float32),
        mesh=mesh,
        scratch_types=[
            pltpu.VMEM((IN_DIM, QROWS), jnp.float32),
            pltpu.VMEM((GCOLS, QROWS), jnp.float32),
            pltpu.VMEM((CH,), jnp.int32),
            pltpu.VMEM((CH,), jnp.float32),
            pltpu.VMEM((GCOLS,), jnp.float32),
            pltpu.VMEM((16,), jnp.int32),
        ],
        compiler_params=pltpu.CompilerParams(use_tc_tiling_on_sc=False,
                                             needs_layout_passes=False),
        interpret=interpret,
    )(_sc_body)


def kernel(x, weights, bias, nonzero_indices):
    nnz = weights.shape[0]
    r = nonzero_indices[:, 0].astype(jnp.int32)
    c = nonzero_indices[:, 1].astype(jnp.int32)
    rkey = r * OUT_DIM + c
    rks, ws = lax.sort((rkey, weights), num_keys=1, is_stable=False)
    keep = jnp.concatenate([rks[1:] != rks[:-1], jnp.ones((1,), bool)])
    ws = jnp.where(keep, ws, 0.0)
    ckey = jnp.bitwise_and(rks, OUT_DIM - 1) * IN_DIM + jnp.right_shift(rks, 12)
    ks, ws = lax.sort((ckey, ws), num_keys=1, is_stable=False)
    offs = jnp.searchsorted(ks, jnp.arange(0, NG + 1, dtype=jnp.int32) * (GCOLS * IN_DIM))
    offs16 = jnp.zeros((16,), jnp.int32).at[:NG + 1].set(offs.astype(jnp.int32))
    nnz_pad = ((nnz + CH - 1) // CH) * CH
    padn = nnz_pad - nnz
    ks = jnp.pad(ks, (0, padn))
    ws = jnp.pad(ws, (0, padn))
    out_t = _make_sc_kernel()(ks, ws, x.T, bias, offs16)
    return out_t.T


# speedup vs baseline: 3.9425x; 1.3599x over previous
import functools

import jax
import jax.numpy as jnp
from jax import lax
from jax.experimental import pallas as pl
from jax.experimental.pallas import tpu as pltpu
from jax.experimental.pallas import tpu_sc as plsc

IN_DIM = 4096
OUT_DIM = 4096
BATCH = 64
NQ = 4
NG = 8
GROWS = IN_DIM // NG
QROWS = BATCH // NQ
CH = 2048
LANES = 16


def _sc_body(k_hbm, w_hbm, xt_hbm, z_hbm, off_hbm, out_hbm,
             xqt, acc, kb, wbuf, offv):
    cidx = lax.axis_index("c")
    sidx = lax.axis_index("s")
    wid = sidx * 2 + cidx
    q = wid & 3
    g = wid >> 2
    g512 = g * GROWS

    pltpu.sync_copy(xt_hbm.at[pl.ds(g512, GROWS), pl.ds(q * QROWS, QROWS)], xqt)
    pltpu.sync_copy(z_hbm, acc)
    pltpu.sync_copy(off_hbm, offv)

    iota = lax.iota(jnp.int32, LANES)
    ov = offv[...]
    s0 = jnp.sum(jnp.where(iota == g, ov, 0))
    s1 = jnp.sum(jnp.where(iota == g + 1, ov, 0))
    k0 = s0 // CH
    k1 = (s1 + CH - 1) // CH

    def chunk_body(i, carry):
        pltpu.sync_copy(k_hbm.at[pl.ds(i * CH, CH)], kb)
        pltpu.sync_copy(w_hbm.at[pl.ds(i * CH, CH)], wbuf)
        base_i = i * CH

        @plsc.parallel_loop(0, CH // LANES, unroll=2)
        def group_body(j):
            sl = pl.ds(j * LANES, LANES)
            kv = kb[sl]
            wv = wbuf[sl]
            rv = jnp.right_shift(kv, 12)
            cv = jnp.bitwise_and(kv, OUT_DIM - 1)
            posv = base_i + j * LANES + iota
            valid = (posv >= s0) & (posv < s1)
            wvm = jnp.where(valid, wv, 0.0)
            rlv = jnp.where(valid, rv - g512, 0)
            for l in range(LANES):
                xrow = xqt[rlv[l], :]
                plsc.addupdate(acc.at[cv[l]], xrow * wvm[l])
        return carry
    lax.fori_loop(k0, k1, chunk_body, 0)

    pltpu.sync_copy(acc, out_hbm.at[g, :, pl.ds(q * QROWS, QROWS)])


def _make_sc_kernel(interpret=False):
    mesh = plsc.VectorSubcoreMesh(core_axis_name="c", subcore_axis_name="s")
    return functools.partial(
        pl.kernel,
        out_type=jax.ShapeDtypeStruct((NG, OUT_DIM, BATCH), jnp.float32),
        mesh=mesh,
        scratch_types=[
            pltpu.VMEM((GROWS, QROWS), jnp.float32),
            pltpu.VMEM((OUT_DIM, QROWS), jnp.float32),
            pltpu.VMEM((CH,), jnp.int32),
            pltpu.VMEM((CH,), jnp.float32),
            pltpu.VMEM((16,), jnp.int32),
        ],
        compiler_params=pltpu.CompilerParams(use_tc_tiling_on_sc=False,
                                             needs_layout_passes=False),
        interpret=interpret,
    )(_sc_body)


def kernel(x, weights, bias, nonzero_indices):
    nnz = weights.shape[0]
    r = nonzero_indices[:, 0].astype(jnp.int32)
    c = nonzero_indices[:, 1].astype(jnp.int32)
    rkey = r * OUT_DIM + c
    ks, ws = lax.sort((rkey, weights), num_keys=1, is_stable=False)
    keep = jnp.concatenate([ks[1:] != ks[:-1], jnp.ones((1,), bool)])
    ws = jnp.where(keep, ws, 0.0)
    offs = jnp.searchsorted(ks, jnp.arange(0, NG + 1, dtype=jnp.int32) * (GROWS * OUT_DIM))
    offs16 = jnp.zeros((16,), jnp.int32).at[:NG + 1].set(offs.astype(jnp.int32))
    nnz_pad = ((nnz + CH - 1) // CH) * CH
    padn = nnz_pad - nnz
    ks = jnp.pad(ks, (0, padn))
    ws = jnp.pad(ws, (0, padn))
    zeros = jnp.zeros((OUT_DIM, QROWS), jnp.float32)
    parts = _make_sc_kernel()(ks, ws, x.T, zeros, offs16)
    out_t = parts.sum(axis=0)
    return out_t.T + bias[None, :]


# speedup vs baseline: 4.0438x; 1.0257x over previous
import functools

import jax
import jax.numpy as jnp
from jax import lax
from jax.experimental import pallas as pl
from jax.experimental.pallas import tpu as pltpu
from jax.experimental.pallas import tpu_sc as plsc

IN_DIM = 4096
OUT_DIM = 4096
BATCH = 64
NQ = 4
NG = 8
GROWS = IN_DIM // NG
QROWS = BATCH // NQ
CH = 4096
LANES = 16


def _sc_body(k_hbm, w_hbm, xt_hbm, z_hbm, off_hbm, out_hbm,
             xqt, acc, kb, wbuf, offv):
    cidx = lax.axis_index("c")
    sidx = lax.axis_index("s")
    wid = sidx * 2 + cidx
    q = wid & 3
    g = wid >> 2
    g512 = g * GROWS

    pltpu.sync_copy(xt_hbm.at[pl.ds(g512, GROWS), pl.ds(q * QROWS, QROWS)], xqt)
    pltpu.sync_copy(z_hbm, acc)
    pltpu.sync_copy(off_hbm, offv)

    iota = lax.iota(jnp.int32, LANES)
    ov = offv[...]
    s0 = jnp.sum(jnp.where(iota == g, ov, 0))
    s1 = jnp.sum(jnp.where(iota == g + 1, ov, 0))
    k0 = s0 // CH
    k1 = (s1 + CH - 1) // CH

    def chunk_body(i, carry):
        pltpu.sync_copy(k_hbm.at[pl.ds(i * CH, CH)], kb)
        pltpu.sync_copy(w_hbm.at[pl.ds(i * CH, CH)], wbuf)
        base_i = i * CH

        @plsc.parallel_loop(0, CH // LANES, unroll=2)
        def group_body(j):
            sl = pl.ds(j * LANES, LANES)
            kv = kb[sl]
            wv = wbuf[sl]
            rv = jnp.right_shift(kv, 12)
            cv = jnp.bitwise_and(kv, OUT_DIM - 1)
            posv = base_i + j * LANES + iota
            valid = (posv >= s0) & (posv < s1)
            wvm = jnp.where(valid, wv, 0.0)
            rlv = jnp.where(valid, rv - g512, 0)
            for l in range(LANES):
                xrow = xqt[rlv[l], :]
                plsc.addupdate(acc.at[cv[l]], xrow * wvm[l])
        return carry
    lax.fori_loop(k0, k1, chunk_body, 0)

    pltpu.sync_copy(acc, out_hbm.at[g, :, pl.ds(q * QROWS, QROWS)])


def _make_sc_kernel(interpret=False):
    mesh = plsc.VectorSubcoreMesh(core_axis_name="c", subcore_axis_name="s")
    return functools.partial(
        pl.kernel,
        out_type=jax.ShapeDtypeStruct((NG, OUT_DIM, BATCH), jnp.float32),
        mesh=mesh,
        scratch_types=[
            pltpu.VMEM((GROWS, QROWS), jnp.float32),
            pltpu.VMEM((OUT_DIM, QROWS), jnp.float32),
            pltpu.VMEM((CH,), jnp.int32),
            pltpu.VMEM((CH,), jnp.float32),
            pltpu.VMEM((16,), jnp.int32),
        ],
        compiler_params=pltpu.CompilerParams(use_tc_tiling_on_sc=False,
                                             needs_layout_passes=False),
        interpret=interpret,
    )(_sc_body)


def kernel(x, weights, bias, nonzero_indices):
    nnz = weights.shape[0]
    r = nonzero_indices[:, 0].astype(jnp.int32)
    c = nonzero_indices[:, 1].astype(jnp.int32)
    rkey = r * OUT_DIM + c
    ks, ws = lax.sort((rkey, weights), num_keys=1, is_stable=False)
    keep = jnp.concatenate([ks[1:] != ks[:-1], jnp.ones((1,), bool)])
    ws = jnp.where(keep, ws, 0.0)
    offs = jnp.searchsorted(ks, jnp.arange(0, NG + 1, dtype=jnp.int32) * (GROWS * OUT_DIM))
    offs16 = jnp.zeros((16,), jnp.int32).at[:NG + 1].set(offs.astype(jnp.int32))
    nnz_pad = ((nnz + CH - 1) // CH) * CH
    padn = nnz_pad - nnz
    ks = jnp.pad(ks, (0, padn))
    ws = jnp.pad(ws, (0, padn))
    zeros = jnp.zeros((OUT_DIM, QROWS), jnp.float32)
    parts = _make_sc_kernel()(ks, ws, x.T, zeros, offs16)
    out_t = parts.sum(axis=0)
    return out_t.T + bias[None, :]


# speedup vs baseline: 4.0524x; 1.0021x over previous
import functools

import jax
import jax.numpy as jnp
from jax import lax
from jax.experimental import pallas as pl
from jax.experimental.pallas import tpu as pltpu
from jax.experimental.pallas import tpu_sc as plsc

IN_DIM = 4096
OUT_DIM = 4096
BATCH = 64
NQ = 4
NG = 8
GROWS = IN_DIM // NG
QROWS = BATCH // NQ
CH = 4096
LANES = 16


def _sc_body(k_hbm, w_hbm, xt_hbm, z_hbm, off_hbm, out_hbm,
             xqt, acc, kb, wbuf, offv):
    cidx = lax.axis_index("c")
    sidx = lax.axis_index("s")
    wid = sidx * 2 + cidx
    q = wid & 3
    g = wid >> 2
    g512 = g * GROWS

    pltpu.sync_copy(xt_hbm.at[pl.ds(g512, GROWS), pl.ds(q * QROWS, QROWS)], xqt)
    pltpu.sync_copy(z_hbm, acc)
    pltpu.sync_copy(off_hbm, offv)

    iota = lax.iota(jnp.int32, LANES)
    ov = offv[...]
    s0 = jnp.sum(jnp.where(iota == g, ov, 0))
    s1 = jnp.sum(jnp.where(iota == g + 1, ov, 0))
    k0 = s0 // CH
    k1 = (s1 + CH - 1) // CH

    def chunk_body(i, carry):
        pltpu.sync_copy(k_hbm.at[pl.ds(i * CH, CH)], kb)
        pltpu.sync_copy(w_hbm.at[pl.ds(i * CH, CH)], wbuf)
        base_i = i * CH

        @plsc.parallel_loop(0, CH // LANES, unroll=4)
        def group_body(j):
            sl = pl.ds(j * LANES, LANES)
            kv = kb[sl]
            wv = wbuf[sl]
            rv = jnp.right_shift(kv, 12)
            cv = jnp.bitwise_and(kv, OUT_DIM - 1)
            posv = base_i + j * LANES + iota
            valid = (posv >= s0) & (posv < s1)
            wvm = jnp.where(valid, wv, 0.0)
            rlv = jnp.where(valid, rv - g512, 0)
            for l in range(LANES):
                xrow = xqt[rlv[l], :]
                plsc.addupdate(acc.at[cv[l]], xrow * wvm[l])
        return carry
    lax.fori_loop(k0, k1, chunk_body, 0)

    pltpu.sync_copy(acc, out_hbm.at[g, :, pl.ds(q * QROWS, QROWS)])


def _make_sc_kernel(interpret=False):
    mesh = plsc.VectorSubcoreMesh(core_axis_name="c", subcore_axis_name="s")
    return functools.partial(
        pl.kernel,
        out_type=jax.ShapeDtypeStruct((NG, OUT_DIM, BATCH), jnp.float32),
        mesh=mesh,
        scratch_types=[
            pltpu.VMEM((GROWS, QROWS), jnp.float32),
            pltpu.VMEM((OUT_DIM, QROWS), jnp.float32),
            pltpu.VMEM((CH,), jnp.int32),
            pltpu.VMEM((CH,), jnp.float32),
            pltpu.VMEM((16,), jnp.int32),
        ],
        compiler_params=pltpu.CompilerParams(use_tc_tiling_on_sc=False,
                                             needs_layout_passes=False),
        interpret=interpret,
    )(_sc_body)


def kernel(x, weights, bias, nonzero_indices):
    nnz = weights.shape[0]
    r = nonzero_indices[:, 0].astype(jnp.int32)
    c = nonzero_indices[:, 1].astype(jnp.int32)
    rkey = r * OUT_DIM + c
    ks, ws = lax.sort((rkey, weights), num_keys=1, is_stable=False)
    keep = jnp.concatenate([ks[1:] != ks[:-1], jnp.ones((1,), bool)])
    ws = jnp.where(keep, ws, 0.0)
    offs = jnp.searchsorted(ks, jnp.arange(0, NG + 1, dtype=jnp.int32) * (GROWS * OUT_DIM))
    offs16 = jnp.zeros((16,), jnp.int32).at[:NG + 1].set(offs.astype(jnp.int32))
    nnz_pad = ((nnz + CH - 1) // CH) * CH
    padn = nnz_pad - nnz
    ks = jnp.pad(ks, (0, padn))
    ws = jnp.pad(ws, (0, padn))
    zeros = jnp.zeros((OUT_DIM, QROWS), jnp.float32)
    parts = _make_sc_kernel()(ks, ws, x.T, zeros, offs16)
    out_t = parts.sum(axis=0)
    return out_t.T + bias[None, :]
